# baseline (speedup 1.0000x reference)
import functools

import jax
import jax.numpy as jnp
from jax import lax
from jax.experimental import pallas as pl
from jax.experimental.pallas import tpu as pltpu
from jax.experimental.pallas import tpu_sc as plsc

_MIN_SIZE = 0.01
_MAX_SIZE = 0.999
_NW = 32
_ROWS = 1_000_000
_NFULL = 7812
_MAIN_ROWS = _NFULL * 128
_CB = 36
_BLKA = 3600
_BLKB = _NFULL - _BLKA


def _pair_step(src, dst, off_a, off_b, sv, f_tiny):
    a = src[pl.ds(off_a, 16)]
    b = src[pl.ds(off_b, 16)]
    t = a * a + b * b
    th = 0.5 * t
    bits = plsc.bitcast(t, jnp.int32)
    bits = 0x5F3759DF - lax.shift_right_logical(bits, 1)
    y = plsc.bitcast(bits, jnp.float32)
    y = y * (1.5 - th * (y * y))
    y = y * (1.5 - th * (y * y))
    factor = jnp.where(t >= 1e-24, sv * y, f_tiny)
    dst[pl.ds(off_a, 16)] = a * factor
    dst[pl.ds(off_b, 16)] = b * factor


def _normalize_chunk(bin_, bout, sv):
    f_tiny = sv * 1e12

    def blk(k, carry):
        for m in range(8):
            a = bin_[k, 0, pl.ds(16 * m, 16)]
            b = bin_[k, 1, pl.ds(16 * m, 16)]
            t = a * a + b * b
            th = 0.5 * t
            bits = plsc.bitcast(t, jnp.int32)
            bits = 0x5F3759DF - lax.shift_right_logical(bits, 1)
            y = plsc.bitcast(bits, jnp.float32)
            y = y * (1.5 - th * (y * y))
            y = y * (1.5 - th * (y * y))
            factor = jnp.where(t >= 1e-24, sv * y, f_tiny)
            bout[2 * k, pl.ds(16 * m, 16)] = a * factor
            bout[2 * k + 1, pl.ds(16 * m, 16)] = b * factor
        return carry

    lax.fori_loop(0, _CB, blk, 0)


def _make_sc_call(nblocks, with_tail):
    mesh = plsc.VectorSubcoreMesh(core_axis_name="c", subcore_axis_name="s")

    nchunk = nblocks // _CB
    maxj = -(-nchunk // _NW)
    out_blocks = nblocks + (1 if with_tail else 0)
    out_blocks_pad = -(-out_blocks // 4) * 4
    _B = 3

    @functools.partial(
        pl.kernel,
        out_type=jax.ShapeDtypeStruct((2 * out_blocks_pad, 128), jnp.float32),
        mesh=mesh,
        scratch_types=(
            [pltpu.VMEM((_CB, 2, 128), jnp.float32)] * _B
            + [pltpu.VMEM((2 * _CB, 128), jnp.float32)] * _B
            + [pltpu.VMEM((16,), jnp.float32)]
            + [pltpu.VMEM((128,), jnp.float32)] * 2
            + [pltpu.SemaphoreType.DMA] * (2 * _B)
        ),
        compiler_params=pltpu.CompilerParams(
            needs_layout_passes=False, use_tc_tiling_on_sc=False
        ),
    )
    def run(*args):
        if with_tail:
            w_hbm, tail_hbm, s_hbm, out_hbm, *scr = args
        else:
            w_hbm, s_hbm, out_hbm, *scr = args
        bin_ = scr[0:_B]
        bout = scr[_B : 2 * _B]
        sbuf = scr[2 * _B]
        tin = scr[2 * _B + 1]
        tout = scr[2 * _B + 2]
        si = scr[2 * _B + 3 : 3 * _B + 3]
        so = scr[3 * _B + 3 : 4 * _B + 3]

        wid = lax.axis_index("s") * 2 + lax.axis_index("c")
        pltpu.sync_copy(s_hbm, sbuf)
        sv = jnp.clip(sbuf[...], _MIN_SIZE, _MAX_SIZE)

        def cid(j):
            return j * _NW + wid

        def start_in(j):
            p = j % _B
            off = pl.multiple_of(cid(j) * _CB, _CB)
            pltpu.async_copy(w_hbm.at[pl.ds(off, _CB)], bin_[p], si[p])

        def wait_in(j):
            p = j % _B
            pltpu.make_async_copy(w_hbm.at[pl.ds(0, _CB)], bin_[p], si[p]).wait()

        def start_out(j):
            p = j % _B
            off = pl.multiple_of(cid(j) * 2 * _CB, 2 * _CB)
            pltpu.async_copy(bout[p], out_hbm.at[pl.ds(off, 2 * _CB)], so[p])

        def wait_out(j):
            p = j % _B
            pltpu.make_async_copy(
                bout[p], out_hbm.at[pl.ds(0, 2 * _CB)], so[p]
            ).wait()

        def valid(j):
            return cid(j) < nchunk

        start_in(0)
        if maxj > 1:
            start_in(1)
        for j in range(maxj):
            if j + 2 < maxj:
                if j - 1 >= 0:
                    wait_out(j - 1)

                @pl.when(valid(j + 2))
                def _():
                    start_in(j + 2)

            if j < maxj - 1:
                wait_in(j)
                _normalize_chunk(bin_[j % _B], bout[j % _B], sv)
                start_out(j)
            else:

                @pl.when(valid(j))
                def _():
                    wait_in(j)
                    _normalize_chunk(bin_[j % _B], bout[j % _B], sv)
                    start_out(j)

        if with_tail:
            @pl.when(wid == _NW - 1)
            def _():
                pltpu.sync_copy(tail_hbm, tin)
                f_tiny = sv * 1e12
                for m in range(4):
                    _pair_step(tin, tout, 16 * m, 64 + 16 * m, sv, f_tiny)
                pltpu.sync_copy(tout.at[pl.ds(0, 64)],
                                out_hbm.at[2 * nblocks, pl.ds(0, 64)])
                pltpu.sync_copy(tout.at[pl.ds(64, 64)],
                                out_hbm.at[2 * nblocks + 1, pl.ds(0, 64)])

        wait_out(maxj - 3)
        wait_out(maxj - 2)

        @pl.when(valid(maxj - 1))
        def _():
            wait_out(maxj - 1)

    return run, out_blocks_pad


_sc_call_a, _PADA = _make_sc_call(_BLKA, False)
_sc_call_b, _PADB = _make_sc_call(_BLKB, True)
_RA = _BLKA * 128


def kernel(weight, scale):
    s16 = jnp.broadcast_to(scale, (16,))
    in_a = weight[:_RA].reshape(_BLKA, 128, 2).swapaxes(1, 2)
    in_b = weight[_RA:_MAIN_ROWS].reshape(_BLKB, 128, 2).swapaxes(1, 2)
    tail_in = weight[_MAIN_ROWS:].T.reshape(128)
    o_a = _sc_call_a(in_a, s16)
    o_b = _sc_call_b(in_b, tail_in, s16)
    z_a = o_a.reshape(_PADA, 2, 128).swapaxes(1, 2).reshape(_PADA * 128, 2)
    z_b = o_b.reshape(_PADB, 2, 128).swapaxes(1, 2).reshape(_PADB * 128, 2)
    out = jnp.zeros((_ROWS, 2), jnp.float32)
    out = lax.dynamic_update_slice(out, z_a[:_RA], (0, 0))
    out = lax.dynamic_update_slice(out, z_b[: _ROWS - _RA], (_RA, 0))
    return out


# speedup vs baseline: 1.1692x; 1.1692x over previous
import functools

import jax
import jax.numpy as jnp
from jax import lax
from jax.experimental import pallas as pl
from jax.experimental.pallas import tpu as pltpu
from jax.experimental.pallas import tpu_sc as plsc

_MIN_SIZE = 0.01
_MAX_SIZE = 0.999
_NW = 32
_ROWS = 1_000_000
_NFULL = 7812
_MAIN_ROWS = _NFULL * 128
_OUTR = 2 * (_NFULL + 4)
_CB = 62
_NCHUNK = _NFULL // _CB
_MAXJ = -(-_NCHUNK // _NW)


def _pair_step(src, dst, off_a, off_b, sv, f_tiny):
    a = src[pl.ds(off_a, 16)]
    b = src[pl.ds(off_b, 16)]
    t = a * a + b * b
    th = 0.5 * t
    bits = plsc.bitcast(t, jnp.int32)
    bits = 0x5F3759DF - lax.shift_right_logical(bits, 1)
    y = plsc.bitcast(bits, jnp.float32)
    y = y * (1.5 - th * (y * y))
    y = y * (1.5 - th * (y * y))
    factor = jnp.where(t >= 1e-24, sv * y, f_tiny)
    dst[pl.ds(off_a, 16)] = a * factor
    dst[pl.ds(off_b, 16)] = b * factor


def _normalize_chunk(bin_, bout, sv):
    f_tiny = sv * 1e12

    def blk(k, carry):
        for m in range(8):
            a = bin_[k, 0, pl.ds(16 * m, 16)]
            b = bin_[k, 1, pl.ds(16 * m, 16)]
            t = a * a + b * b
            th = 0.5 * t
            bits = plsc.bitcast(t, jnp.int32)
            bits = 0x5F3759DF - lax.shift_right_logical(bits, 1)
            y = plsc.bitcast(bits, jnp.float32)
            y = y * (1.5 - th * (y * y))
            y = y * (1.5 - th * (y * y))
            factor = jnp.where(t >= 1e-24, sv * y, f_tiny)
            bout[2 * k, pl.ds(16 * m, 16)] = a * factor
            bout[2 * k + 1, pl.ds(16 * m, 16)] = b * factor
        return carry

    lax.fori_loop(0, _CB, blk, 0)


def _make_sc_call():
    mesh = plsc.VectorSubcoreMesh(core_axis_name="c", subcore_axis_name="s")

    _B = 3

    @functools.partial(
        pl.kernel,
        out_type=jax.ShapeDtypeStruct((_OUTR, 128), jnp.float32),
        mesh=mesh,
        scratch_types=(
            [pltpu.VMEM((_CB, 2, 128), jnp.float32)] * _B
            + [pltpu.VMEM((2 * _CB, 128), jnp.float32)] * _B
            + [pltpu.VMEM((16,), jnp.float32)]
            + [pltpu.VMEM((128,), jnp.float32)] * 2
            + [pltpu.SemaphoreType.DMA] * (2 * _B)
        ),
        compiler_params=pltpu.CompilerParams(
            needs_layout_passes=False, use_tc_tiling_on_sc=False
        ),
    )
    def run(w_hbm, tail_hbm, s_hbm, out_hbm, *scr):
        bin_ = scr[0:_B]
        bout = scr[_B : 2 * _B]
        sbuf = scr[2 * _B]
        tin = scr[2 * _B + 1]
        tout = scr[2 * _B + 2]
        si = scr[2 * _B + 3 : 3 * _B + 3]
        so = scr[3 * _B + 3 : 4 * _B + 3]

        wid = lax.axis_index("s") * 2 + lax.axis_index("c")
        pltpu.sync_copy(s_hbm, sbuf)
        sv = jnp.clip(sbuf[...], _MIN_SIZE, _MAX_SIZE)

        def cid(j):
            return j * _NW + wid

        def start_in(j):
            p = j % _B
            off = pl.multiple_of(cid(j) * _CB, _CB)
            pltpu.async_copy(w_hbm.at[pl.ds(off, _CB)], bin_[p], si[p])

        def wait_in(j):
            p = j % _B
            pltpu.make_async_copy(w_hbm.at[pl.ds(0, _CB)], bin_[p], si[p]).wait()

        def start_out(j):
            p = j % _B
            off = pl.multiple_of(cid(j) * 2 * _CB, 2 * _CB)
            pltpu.async_copy(bout[p], out_hbm.at[pl.ds(off, 2 * _CB)], so[p])

        def wait_out(j):
            p = j % _B
            pltpu.make_async_copy(
                bout[p], out_hbm.at[pl.ds(0, 2 * _CB)], so[p]
            ).wait()

        def valid(j):
            return cid(j) < _NCHUNK

        start_in(0)
        if _MAXJ > 1:
            start_in(1)
        for j in range(_MAXJ):
            if j + 2 < _MAXJ:
                if j - 1 >= 0:
                    wait_out(j - 1)

                @pl.when(valid(j + 2))
                def _():
                    start_in(j + 2)

            if j < _MAXJ - 1:
                wait_in(j)
                _normalize_chunk(bin_[j % _B], bout[j % _B], sv)
                start_out(j)
            else:

                @pl.when(valid(j))
                def _():
                    wait_in(j)
                    _normalize_chunk(bin_[j % _B], bout[j % _B], sv)
                    start_out(j)

        @pl.when(wid == _NW - 1)
        def _():
            pltpu.sync_copy(tail_hbm, tin)
            f_tiny = sv * 1e12
            for m in range(4):
                _pair_step(tin, tout, 16 * m, 64 + 16 * m, sv, f_tiny)
            pltpu.sync_copy(tout.at[pl.ds(0, 64)],
                            out_hbm.at[2 * _NFULL, pl.ds(0, 64)])
            pltpu.sync_copy(tout.at[pl.ds(64, 64)],
                            out_hbm.at[2 * _NFULL + 1, pl.ds(0, 64)])

        wait_out(_MAXJ - 3)
        wait_out(_MAXJ - 2)

        @pl.when(valid(_MAXJ - 1))
        def _():
            wait_out(_MAXJ - 1)

    return run


_sc_call = _make_sc_call()


def kernel(weight, scale):
    s16 = jnp.broadcast_to(scale, (16,))
    in3 = weight[:_MAIN_ROWS].reshape(_NFULL, 128, 2).swapaxes(1, 2)
    tail_in = weight[_MAIN_ROWS:].T.reshape(128)
    o = _sc_call(in3, tail_in, s16)
    z = (
        o.reshape(_NFULL + 4, 2, 128)
        .swapaxes(1, 2)
        .reshape((_NFULL + 4) * 128, 2)
    )
    return z[:_ROWS]
